# baseline (speedup 1.0000x reference)
import functools

import jax
import jax.numpy as jnp
from jax import lax
from jax.experimental import pallas as pl
from jax.experimental.pallas import tpu as pltpu
from jax.experimental.pallas import tpu_sc as plsc

B, S = 8, 576
N = B * S
IN_D = 768
HID = 256
OUT_D = 768

TOK_BLK = 1536
CHUNK0 = 1536
CHUNK1 = N - CHUNK0
BLKS0 = CHUNK0 // TOK_BLK
BLKS1 = CHUNK1 // TOK_BLK

_NC = 2
_NS = 16
_NW = _NC * _NS
_ROWS_W0 = CHUNK0 // _NW
_ROWS_W1 = CHUNK1 // _NW


def _enc_chunk0_kernel(x_ref, w_ref, b_ref, cb_ref, dw_ref, db_ref,
                       idx_ref, table_ref):
    ht = lax.dot_general(w_ref[...], x_ref[...],
                         (((1,), (1,)), ((), ())),
                         preferred_element_type=jnp.float32)
    ht = ht + jnp.transpose(jnp.reshape(b_ref[...], (1, HID)))
    m = jnp.max(ht, axis=0, keepdims=True)
    ii = lax.broadcasted_iota(jnp.int32, ht.shape, 0)
    idx = jnp.min(jnp.where(ht == m, ii, HID), axis=0)
    idx_ref[...] = idx.astype(jnp.int32).reshape(1, 1, TOK_BLK)

    @pl.when(pl.program_id(0) == 0)
    def _():
        t = lax.dot_general(cb_ref[...], dw_ref[...],
                            (((1,), (1,)), ((), ())),
                            preferred_element_type=jnp.float32)
        table_ref[...] = (t + db_ref[...][None, :])[:, None, :]


def _enc_chunk_kernel(x_ref, w_ref, b_ref, idx_ref):
    ht = lax.dot_general(w_ref[...], x_ref[...],
                         (((1,), (1,)), ((), ())),
                         preferred_element_type=jnp.float32)
    ht = ht + jnp.transpose(jnp.reshape(b_ref[...], (1, HID)))
    m = jnp.max(ht, axis=0, keepdims=True)
    ii = lax.broadcasted_iota(jnp.int32, ht.shape, 0)
    idx = jnp.min(jnp.where(ht == m, ii, HID), axis=0)
    idx_ref[...] = idx.astype(jnp.int32).reshape(1, 1, TOK_BLK)


def _enc_chunk0(xf, enc_w, enc_b, codebook, dec_w, dec_b):
    return pl.pallas_call(
        _enc_chunk0_kernel,
        grid=(BLKS0,),
        in_specs=[
            pl.BlockSpec((TOK_BLK, IN_D), lambda i: (i, 0)),
            pl.BlockSpec((HID, IN_D), lambda i: (0, 0)),
            pl.BlockSpec((HID,), lambda i: (0,)),
            pl.BlockSpec((HID, HID), lambda i: (0, 0)),
            pl.BlockSpec((OUT_D, HID), lambda i: (0, 0)),
            pl.BlockSpec((OUT_D,), lambda i: (0,)),
        ],
        out_specs=[
            pl.BlockSpec((1, 1, TOK_BLK), lambda i: (i, 0, 0)),
            pl.BlockSpec((HID, 1, OUT_D), lambda i: (0, 0, 0)),
        ],
        out_shape=[
            jax.ShapeDtypeStruct((BLKS0, 1, TOK_BLK), jnp.int32),
            jax.ShapeDtypeStruct((HID, 1, OUT_D), jnp.float32),
        ],
    )(xf, enc_w, enc_b, codebook, dec_w, dec_b)


def _enc_chunk1(xf, enc_w, enc_b):
    return pl.pallas_call(
        _enc_chunk_kernel,
        grid=(BLKS1,),
        in_specs=[
            pl.BlockSpec((TOK_BLK, IN_D), lambda i: (BLKS0 + i, 0)),
            pl.BlockSpec((HID, IN_D), lambda i: (0, 0)),
            pl.BlockSpec((HID,), lambda i: (0,)),
        ],
        out_specs=pl.BlockSpec((1, 1, TOK_BLK), lambda i: (i, 0, 0)),
        out_shape=jax.ShapeDtypeStruct((BLKS1, 1, TOK_BLK), jnp.int32),
    )(xf, enc_w, enc_b)


@functools.cache
def _make_sc_head():
    @functools.partial(
        pl.kernel,
        mesh=plsc.VectorSubcoreMesh(core_axis_name="c", subcore_axis_name="s"),
        out_type=jax.ShapeDtypeStruct((N, 1, OUT_D), jnp.float32),
        scratch_types=[
            pltpu.VMEM((_ROWS_W0,), jnp.int32),
            pltpu.VMEM((_ROWS_W0, 1, OUT_D), jnp.float32),
            pltpu.SemaphoreType.DMA,
        ],
    )
    def _sc_head(table_hbm, idx_hbm, out_hbm, idx_v, rows_v, sem):
        wid = lax.axis_index("s") * _NC + lax.axis_index("c")
        base = wid * _ROWS_W0
        pltpu.sync_copy(idx_hbm.at[pl.ds(base, _ROWS_W0)], idx_v)
        pltpu.async_copy(table_hbm.at[idx_v], rows_v, sem).wait()
        pltpu.sync_copy(rows_v, out_hbm.at[pl.ds(base, _ROWS_W0)])

    return _sc_head


@functools.cache
def _make_sc_tail():
    @functools.partial(
        pl.kernel,
        mesh=plsc.VectorSubcoreMesh(core_axis_name="c", subcore_axis_name="s"),
        out_type=(),
        scratch_types=[
            pltpu.VMEM((_ROWS_W1,), jnp.int32),
            pltpu.VMEM((_ROWS_W1, 1, OUT_D), jnp.float32),
            pltpu.SemaphoreType.DMA,
        ],
    )
    def _sc_tail(table_hbm, idx_hbm, out_hbm, idx_v, rows_v, sem):
        wid = lax.axis_index("s") * _NC + lax.axis_index("c")
        base = wid * _ROWS_W1
        pltpu.sync_copy(idx_hbm.at[pl.ds(base, _ROWS_W1)], idx_v)
        pltpu.async_copy(table_hbm.at[idx_v], rows_v, sem).wait()
        pltpu.sync_copy(rows_v, out_hbm.at[pl.ds(CHUNK0 + base, _ROWS_W1)])

    return _sc_tail


def kernel(x, enc_w, enc_b, dec_w, dec_b, codebook):
    xf = x.reshape(N, IN_D)
    idx0, table = _enc_chunk0(xf, enc_w, enc_b, codebook, dec_w, dec_b)
    idx1 = _enc_chunk1(xf, enc_w, enc_b)
    idx0 = idx0.reshape(CHUNK0)
    idx1 = idx1.reshape(CHUNK1)
    out0 = _make_sc_head()(table, idx0)
    out_ref = jax.new_ref(out0)
    _make_sc_tail()(table, idx1, out_ref)
    return out_ref[...].reshape(B, S, 1, OUT_D)


# speedup vs baseline: 1.0010x; 1.0010x over previous
import functools

import jax
import jax.numpy as jnp
from jax import lax
from jax.experimental import pallas as pl
from jax.experimental.pallas import tpu as pltpu
from jax.experimental.pallas import tpu_sc as plsc

B, S = 8, 576
N = B * S
IN_D = 768
HID = 256
OUT_D = 768

TOK_BLK = 512
CHUNK0 = 2048
CHUNK1 = N - CHUNK0
BLKS0 = CHUNK0 // TOK_BLK
BLKS1 = CHUNK1 // TOK_BLK

_NC = 2
_NS = 16
_NW = _NC * _NS
_ROWS_W0 = CHUNK0 // _NW
_ROWS_W1 = CHUNK1 // _NW


def _enc_chunk0_kernel(x_ref, w_ref, b_ref, cb_ref, dw_ref, db_ref,
                       idx_ref, table_ref):
    ht = lax.dot_general(w_ref[...], x_ref[...],
                         (((1,), (1,)), ((), ())),
                         preferred_element_type=jnp.float32)
    ht = ht + jnp.transpose(jnp.reshape(b_ref[...], (1, HID)))
    m = jnp.max(ht, axis=0, keepdims=True)
    ii = lax.broadcasted_iota(jnp.int32, ht.shape, 0)
    idx = jnp.min(jnp.where(ht == m, ii, HID), axis=0)
    idx_ref[...] = idx.astype(jnp.int32).reshape(1, 1, TOK_BLK)

    @pl.when(pl.program_id(0) == 0)
    def _():
        t = lax.dot_general(cb_ref[...], dw_ref[...],
                            (((1,), (1,)), ((), ())),
                            preferred_element_type=jnp.float32)
        table_ref[...] = (t + db_ref[...][None, :])[:, None, :]


def _enc_chunk_kernel(x_ref, w_ref, b_ref, idx_ref):
    ht = lax.dot_general(w_ref[...], x_ref[...],
                         (((1,), (1,)), ((), ())),
                         preferred_element_type=jnp.float32)
    ht = ht + jnp.transpose(jnp.reshape(b_ref[...], (1, HID)))
    m = jnp.max(ht, axis=0, keepdims=True)
    ii = lax.broadcasted_iota(jnp.int32, ht.shape, 0)
    idx = jnp.min(jnp.where(ht == m, ii, HID), axis=0)
    idx_ref[...] = idx.astype(jnp.int32).reshape(1, 1, TOK_BLK)


def _enc_chunk0(xf, enc_w, enc_b, codebook, dec_w, dec_b):
    return pl.pallas_call(
        _enc_chunk0_kernel,
        grid=(BLKS0,),
        in_specs=[
            pl.BlockSpec((TOK_BLK, IN_D), lambda i: (i, 0)),
            pl.BlockSpec((HID, IN_D), lambda i: (0, 0)),
            pl.BlockSpec((HID,), lambda i: (0,)),
            pl.BlockSpec((HID, HID), lambda i: (0, 0)),
            pl.BlockSpec((OUT_D, HID), lambda i: (0, 0)),
            pl.BlockSpec((OUT_D,), lambda i: (0,)),
        ],
        out_specs=[
            pl.BlockSpec((1, 1, TOK_BLK), lambda i: (i, 0, 0)),
            pl.BlockSpec((HID, 1, OUT_D), lambda i: (0, 0, 0)),
        ],
        out_shape=[
            jax.ShapeDtypeStruct((BLKS0, 1, TOK_BLK), jnp.int32),
            jax.ShapeDtypeStruct((HID, 1, OUT_D), jnp.float32),
        ],
    )(xf, enc_w, enc_b, codebook, dec_w, dec_b)


def _enc_chunk1(xf, enc_w, enc_b):
    return pl.pallas_call(
        _enc_chunk_kernel,
        grid=(BLKS1,),
        in_specs=[
            pl.BlockSpec((TOK_BLK, IN_D), lambda i: (BLKS0 + i, 0)),
            pl.BlockSpec((HID, IN_D), lambda i: (0, 0)),
            pl.BlockSpec((HID,), lambda i: (0,)),
        ],
        out_specs=pl.BlockSpec((1, 1, TOK_BLK), lambda i: (i, 0, 0)),
        out_shape=jax.ShapeDtypeStruct((BLKS1, 1, TOK_BLK), jnp.int32),
    )(xf, enc_w, enc_b)


@functools.cache
def _make_sc_head():
    @functools.partial(
        pl.kernel,
        mesh=plsc.VectorSubcoreMesh(core_axis_name="c", subcore_axis_name="s"),
        out_type=jax.ShapeDtypeStruct((N, 1, OUT_D), jnp.float32),
        scratch_types=[
            pltpu.VMEM((_ROWS_W0,), jnp.int32),
            pltpu.VMEM((_ROWS_W0, 1, OUT_D), jnp.float32),
            pltpu.SemaphoreType.DMA,
        ],
    )
    def _sc_head(table_hbm, idx_hbm, out_hbm, idx_v, rows_v, sem):
        wid = lax.axis_index("s") * _NC + lax.axis_index("c")
        base = wid * _ROWS_W0
        pltpu.sync_copy(idx_hbm.at[pl.ds(base, _ROWS_W0)], idx_v)
        pltpu.async_copy(table_hbm.at[idx_v], rows_v, sem).wait()
        pltpu.sync_copy(rows_v, out_hbm.at[pl.ds(base, _ROWS_W0)])

    return _sc_head


@functools.cache
def _make_sc_tail():
    @functools.partial(
        pl.kernel,
        mesh=plsc.VectorSubcoreMesh(core_axis_name="c", subcore_axis_name="s"),
        out_type=(),
        scratch_types=[
            pltpu.VMEM((_ROWS_W1,), jnp.int32),
            pltpu.VMEM((_ROWS_W1, 1, OUT_D), jnp.float32),
            pltpu.SemaphoreType.DMA,
        ],
    )
    def _sc_tail(table_hbm, idx_hbm, out_hbm, idx_v, rows_v, sem):
        wid = lax.axis_index("s") * _NC + lax.axis_index("c")
        base = wid * _ROWS_W1
        pltpu.sync_copy(idx_hbm.at[pl.ds(base, _ROWS_W1)], idx_v)
        pltpu.async_copy(table_hbm.at[idx_v], rows_v, sem).wait()
        pltpu.sync_copy(rows_v, out_hbm.at[pl.ds(CHUNK0 + base, _ROWS_W1)])

    return _sc_tail


def kernel(x, enc_w, enc_b, dec_w, dec_b, codebook):
    xf = x.reshape(N, IN_D)
    idx0, table = _enc_chunk0(xf, enc_w, enc_b, codebook, dec_w, dec_b)
    idx1 = _enc_chunk1(xf, enc_w, enc_b)
    idx0 = idx0.reshape(CHUNK0)
    idx1 = idx1.reshape(CHUNK1)
    out0 = _make_sc_head()(table, idx0)
    out_ref = jax.new_ref(out0)
    _make_sc_tail()(table, idx1, out_ref)
    return out_ref[...].reshape(B, S, 1, OUT_D)


# speedup vs baseline: 1.0157x; 1.0147x over previous
import functools

import jax
import jax.numpy as jnp
from jax import lax
from jax.experimental import pallas as pl
from jax.experimental.pallas import tpu as pltpu
from jax.experimental.pallas import tpu_sc as plsc

B, S = 8, 576
N = B * S
IN_D = 768
HID = 256
OUT_D = 768

TOK_BLK = 512
CHUNK0 = 2048
CHUNK1 = N - CHUNK0
BLKS0 = CHUNK0 // TOK_BLK
BLKS1 = CHUNK1 // TOK_BLK

_NC = 2
_NS = 16
_NW = _NC * _NS
_ROWS_W0 = CHUNK0 // _NW
_ROWS_W1 = CHUNK1 // _NW


def _enc_chunk0_kernel(x_ref, w_ref, b_ref, cb_ref, dw_ref, db_ref,
                       idx_ref, table_ref):
    ht = lax.dot_general(w_ref[...], x_ref[...],
                         (((1,), (1,)), ((), ())),
                         preferred_element_type=jnp.float32)
    ht = ht + jnp.transpose(jnp.reshape(b_ref[...], (1, HID)))
    m = jnp.max(ht, axis=0, keepdims=True)
    ii = lax.broadcasted_iota(jnp.int32, ht.shape, 0)
    idx = jnp.min(jnp.where(ht == m, ii, HID), axis=0)
    idx_ref[...] = idx.astype(jnp.int32)

    @pl.when(pl.program_id(0) == 0)
    def _():
        t = lax.dot_general(cb_ref[...], dw_ref[...],
                            (((1,), (1,)), ((), ())),
                            preferred_element_type=jnp.float32)
        table_ref[...] = (t + db_ref[...][None, :])[:, None, :]


def _enc_chunk_kernel(x_ref, w_ref, b_ref, idx_ref):
    ht = lax.dot_general(w_ref[...], x_ref[...],
                         (((1,), (1,)), ((), ())),
                         preferred_element_type=jnp.float32)
    ht = ht + jnp.transpose(jnp.reshape(b_ref[...], (1, HID)))
    m = jnp.max(ht, axis=0, keepdims=True)
    ii = lax.broadcasted_iota(jnp.int32, ht.shape, 0)
    idx = jnp.min(jnp.where(ht == m, ii, HID), axis=0)
    idx_ref[...] = idx.astype(jnp.int32)


def _enc_chunk0(xf, enc_w, enc_b, codebook, dec_w, dec_b):
    return pl.pallas_call(
        _enc_chunk0_kernel,
        grid=(BLKS0,),
        in_specs=[
            pl.BlockSpec((TOK_BLK, IN_D), lambda i: (i, 0)),
            pl.BlockSpec((HID, IN_D), lambda i: (0, 0)),
            pl.BlockSpec((HID,), lambda i: (0,)),
            pl.BlockSpec((HID, HID), lambda i: (0, 0)),
            pl.BlockSpec((OUT_D, HID), lambda i: (0, 0)),
            pl.BlockSpec((OUT_D,), lambda i: (0,)),
        ],
        out_specs=[
            pl.BlockSpec((TOK_BLK,), lambda i: (i,)),
            pl.BlockSpec((HID, 1, OUT_D), lambda i: (0, 0, 0)),
        ],
        out_shape=[
            jax.ShapeDtypeStruct((CHUNK0,), jnp.int32),
            jax.ShapeDtypeStruct((HID, 1, OUT_D), jnp.float32),
        ],
    )(xf, enc_w, enc_b, codebook, dec_w, dec_b)


def _enc_chunk1(xf, enc_w, enc_b):
    return pl.pallas_call(
        _enc_chunk_kernel,
        grid=(BLKS1,),
        in_specs=[
            pl.BlockSpec((TOK_BLK, IN_D), lambda i: (BLKS0 + i, 0)),
            pl.BlockSpec((HID, IN_D), lambda i: (0, 0)),
            pl.BlockSpec((HID,), lambda i: (0,)),
        ],
        out_specs=pl.BlockSpec((TOK_BLK,), lambda i: (i,)),
        out_shape=jax.ShapeDtypeStruct((CHUNK1,), jnp.int32),
    )(xf, enc_w, enc_b)


@functools.cache
def _make_sc_head():
    @functools.partial(
        pl.kernel,
        mesh=plsc.VectorSubcoreMesh(core_axis_name="c", subcore_axis_name="s"),
        out_type=jax.ShapeDtypeStruct((N, 1, OUT_D), jnp.float32),
        scratch_types=[
            pltpu.VMEM((_ROWS_W0,), jnp.int32),
            pltpu.VMEM((_ROWS_W0, 1, OUT_D), jnp.float32),
            pltpu.SemaphoreType.DMA,
        ],
    )
    def _sc_head(table_hbm, idx_hbm, out_hbm, idx_v, rows_v, sem):
        wid = lax.axis_index("s") * _NC + lax.axis_index("c")
        base = wid * _ROWS_W0
        pltpu.sync_copy(idx_hbm.at[pl.ds(base, _ROWS_W0)], idx_v)
        pltpu.async_copy(table_hbm.at[idx_v], rows_v, sem).wait()
        pltpu.sync_copy(rows_v, out_hbm.at[pl.ds(base, _ROWS_W0)])

    return _sc_head


@functools.cache
def _make_sc_tail():
    @functools.partial(
        pl.kernel,
        mesh=plsc.VectorSubcoreMesh(core_axis_name="c", subcore_axis_name="s"),
        out_type=(),
        scratch_types=[
            pltpu.VMEM((_ROWS_W1,), jnp.int32),
            pltpu.VMEM((_ROWS_W1, 1, OUT_D), jnp.float32),
            pltpu.SemaphoreType.DMA,
        ],
    )
    def _sc_tail(table_hbm, idx_hbm, out_hbm, idx_v, rows_v, sem):
        wid = lax.axis_index("s") * _NC + lax.axis_index("c")
        base = wid * _ROWS_W1
        pltpu.sync_copy(idx_hbm.at[pl.ds(base, _ROWS_W1)], idx_v)
        pltpu.async_copy(table_hbm.at[idx_v], rows_v, sem).wait()
        pltpu.sync_copy(rows_v, out_hbm.at[pl.ds(CHUNK0 + base, _ROWS_W1)])

    return _sc_tail


def kernel(x, enc_w, enc_b, dec_w, dec_b, codebook):
    xf = x.reshape(N, IN_D)
    idx0, table = _enc_chunk0(xf, enc_w, enc_b, codebook, dec_w, dec_b)
    idx1 = _enc_chunk1(xf, enc_w, enc_b)
    out0 = _make_sc_head()(table, idx0)
    out_ref = jax.new_ref(out0)
    _make_sc_tail()(table, idx1, out_ref)
    return out_ref[...].reshape(B, S, 1, OUT_D)
